# baseline (speedup 1.0000x reference)
import jax
import jax.numpy as jnp
from jax import lax
from jax.experimental import pallas as pl
from jax.experimental.pallas import tpu as pltpu
from jax.experimental.pallas import tpu_sc as plsc

B = 16
H_KV = 2
MAX_LEN = 2048
HEAD_DIM = 128
NW = 32
ROWS = 4 * B * H_KV * MAX_LEN

NBUF = 4
CHUNK = MAX_LEN


def _copy_body(c0, c1, c2, c3, out, shared, sem_in, sem_out):
    cid = lax.axis_index("c")
    caches = (c0, c1, c2, c3)
    ids = [(c, bh) for c in range(4) for bh in range(16)]

    def src_slice(t):
        c, bh = t
        return caches[c].at[pl.ds((cid * 16 + bh) * MAX_LEN, CHUNK)]

    def dst_slice(t):
        c, bh = t
        return out.at[pl.ds((c * NW + cid * 16 + bh) * MAX_LEN, CHUNK)]

    n = len(ids)
    in_cp = [None] * NBUF
    out_cp = [None] * NBUF
    for j in range(NBUF):
        in_cp[j] = pltpu.async_copy(src_slice(ids[j]), shared.at[j],
                                    sem_in.at[j])
    for i in range(n):
        j = i % NBUF
        in_cp[j].wait()
        out_cp[j] = pltpu.async_copy(shared.at[j], dst_slice(ids[i]),
                                     sem_out.at[j])
        nxt = i + NBUF
        if nxt < n:
            out_cp[j].wait()
            in_cp[j] = pltpu.async_copy(src_slice(ids[nxt]), shared.at[j],
                                        sem_in.at[j])
    for i in range(n - NBUF, n):
        out_cp[i % NBUF].wait()


def _scatter_body(pos_ref, n0, n1, n2, n3, out_in, out_ref, sem):
    del out_in
    news = (n0, n1, n2, n3)
    copies = []
    for c in range(4):
        for b in range(B):
            pos_b = pos_ref[b]
            for h in range(H_KV):
                copies.append(pltpu.make_async_copy(
                    news[c].at[b, h],
                    out_ref.at[c, b, h].at[pl.ds(pos_b, 1)],
                    sem))
    for cp in copies:
        cp.start()
    for cp in copies:
        cp.wait()


def kernel(k_cache_0, v_cache_0, k_cache_1, v_cache_1,
           new_k_0, new_v_0, new_k_1, new_v_1,
           position_ids, seq_ids):
    del seq_ids
    pos = position_ids[:, 0].astype(jnp.int32)

    flat = lambda c: c.reshape(B * H_KV * MAX_LEN, HEAD_DIM)
    mesh = plsc.ScalarSubcoreMesh(axis_name="c", num_cores=2)
    copied = pl.kernel(
        _copy_body,
        out_type=jax.ShapeDtypeStruct((ROWS, HEAD_DIM), jnp.float32),
        mesh=mesh,
        scratch_types=[
            pltpu.VMEM_SHARED((NBUF, CHUNK, HEAD_DIM), jnp.float32),
            pltpu.SemaphoreType.DMA((NBUF,)),
            pltpu.SemaphoreType.DMA((NBUF,)),
        ],
    )(flat(k_cache_0), flat(v_cache_0), flat(k_cache_1), flat(v_cache_1))
    copied = copied.reshape(4, B, H_KV, MAX_LEN, HEAD_DIM)

    hbm_spec = pl.BlockSpec(memory_space=pltpu.MemorySpace.HBM)
    grid_spec = pltpu.PrefetchScalarGridSpec(
        num_scalar_prefetch=1,
        grid=(),
        in_specs=[hbm_spec] * 5,
        out_specs=hbm_spec,
        scratch_shapes=[pltpu.SemaphoreType.DMA],
    )
    return pl.pallas_call(
        _scatter_body,
        grid_spec=grid_spec,
        out_shape=jax.ShapeDtypeStruct((4, B, H_KV, MAX_LEN, HEAD_DIM),
                                       jnp.float32),
        input_output_aliases={5: 0},
    )(pos, new_k_0, new_v_0, new_k_1, new_v_1, copied)


# speedup vs baseline: 1.2359x; 1.2359x over previous
import jax
import jax.numpy as jnp
from jax import lax
from jax.experimental import pallas as pl
from jax.experimental.pallas import tpu as pltpu
from jax.experimental.pallas import tpu_sc as plsc

B = 16
H_KV = 2
MAX_LEN = 2048
HEAD_DIM = 128
NW = 32
ROWS = 4 * B * H_KV * MAX_LEN


CHUNK = 256
NBUF = 3
NCHUNK = 4 * MAX_LEN // CHUNK


class _Ring:
    def __init__(self, bufs, sem_in, sem_out, chunk_ids, src_slice, dst_slice):
        self.bufs = bufs
        self.sem_in = sem_in
        self.sem_out = sem_out
        self.ids = chunk_ids
        self.src = src_slice
        self.dst = dst_slice
        self.n = len(chunk_ids)
        self.nbuf = len(bufs)
        self.in_cp = [None] * self.nbuf
        self.out_cp = [None] * self.nbuf

    def prime(self):
        for j in range(min(self.nbuf, self.n)):
            self.in_cp[j] = pltpu.async_copy(
                self.src(self.ids[j]), self.bufs[j], self.sem_in.at[j])

    def step(self, i):
        if i >= self.n:
            return
        j = i % self.nbuf
        self.in_cp[j].wait()
        self.out_cp[j] = pltpu.async_copy(
            self.bufs[j], self.dst(self.ids[i]), self.sem_out.at[j])
        nxt = i + self.nbuf
        if nxt < self.n:
            self.out_cp[j].wait()
            self.in_cp[j] = pltpu.async_copy(
                self.src(self.ids[nxt]), self.bufs[j], self.sem_in.at[j])

    def drain(self):
        for i in range(max(0, self.n - self.nbuf), self.n):
            self.out_cp[i % self.nbuf].wait()


def _body(c0, c1, c2, c3, rows_hbm, idx_hbm, out,
          shared, idx_v, rows_v, sem_in, sem_out, sem_row, sem_pre):
    s = lax.axis_index("s")
    w = s * 2 + lax.axis_index("c")
    caches = (c0, c1, c2, c3)

    def src_slice(i):
        c, k = divmod(i, MAX_LEN // CHUNK)
        return caches[c].at[pl.ds(w * MAX_LEN + k * CHUNK, CHUNK)]

    def dst_slice(i):
        c, k = divmod(i, MAX_LEN // CHUNK)
        return out.at[pl.ds((c * NW + w) * MAX_LEN + k * CHUNK, CHUNK)]

    pre_idx = pltpu.async_copy(idx_hbm.at[w], idx_v, sem_pre)
    pre_rows = pltpu.async_copy(rows_hbm.at[w], rows_v, sem_pre)

    order = [c * (MAX_LEN // CHUNK) + k
             for k in range(MAX_LEN // CHUNK) for c in range(4)]
    ring = _Ring(tuple(shared.at[s, j] for j in range(NBUF)),
                 sem_in, sem_out, order, src_slice, dst_slice)
    ring.prime()
    for i in range(NCHUNK):
        ring.step(i)
    ring.drain()
    pre_idx.wait()
    pre_rows.wait()
    pltpu.async_copy(rows_v, out.at[idx_v], sem_row).wait()


def kernel(k_cache_0, v_cache_0, k_cache_1, v_cache_1,
           new_k_0, new_v_0, new_k_1, new_v_1,
           position_ids, seq_ids):
    del seq_ids
    pos = position_ids[:, 0].astype(jnp.int32)

    flat = lambda c: c.reshape(B * H_KV * MAX_LEN, HEAD_DIM)
    new_rows = jnp.stack(
        [new_k_0[:, :, 0], new_v_0[:, :, 0], new_k_1[:, :, 0], new_v_1[:, :, 0]],
        axis=2,
    ).reshape(B * H_KV, 4, HEAD_DIM)
    bh = jnp.arange(B * H_KV, dtype=jnp.int32)
    c = jnp.arange(4, dtype=jnp.int32)
    dest_idx = (c[None, :] * NW + bh[:, None]) * MAX_LEN + pos[bh // H_KV][:, None]

    mesh = plsc.VectorSubcoreMesh(core_axis_name="c", subcore_axis_name="s")
    out = pl.kernel(
        _body,
        out_type=jax.ShapeDtypeStruct((ROWS, HEAD_DIM), jnp.float32),
        mesh=mesh,
        scratch_types=[
            pltpu.VMEM_SHARED((16, NBUF, CHUNK, HEAD_DIM), jnp.float32),
            pltpu.VMEM((4,), jnp.int32),
            pltpu.VMEM((4, HEAD_DIM), jnp.float32),
            pltpu.SemaphoreType.DMA((NBUF,)),
            pltpu.SemaphoreType.DMA((NBUF,)),
            pltpu.SemaphoreType.DMA,
            pltpu.SemaphoreType.DMA,
        ],
    )(flat(k_cache_0), flat(v_cache_0), flat(k_cache_1), flat(v_cache_1),
      new_rows, dest_idx)
    return out.reshape(4, B, H_KV, MAX_LEN, HEAD_DIM)
